# Initial kernel scaffold; baseline (speedup 1.0000x reference)
import jax
import jax.numpy as jnp
from jax.experimental import pallas as pl


def kernel(sp, coord, veloc, t, Wq, mkW1, mkb1, mkW2, mkb2, mvW1, mvb1, mvW2, mvb2, w_dot, Wout0e, Wout1o):
    raise NotImplementedError("write your pallas kernel here")


# speedup vs baseline: 5.1688x; 5.1688x over previous
import functools

import jax
import jax.numpy as jnp
import numpy as np
from jax.experimental import pallas as pl

B, N, H = 2, 128, 4
SP, TD, DE = 16, 8, 16
CUT = 5.0
DK = 8
DV0, DV1 = 8, 2

RSTEP = np.float32(CUT / (DE - 1))
SQRT26 = np.float32(np.sqrt(26.0))
SQRT78 = np.float32(np.sqrt(3.0) * np.sqrt(26.0))
ASCL = np.float32(1.0 / (np.sqrt(64.0) * np.sqrt(float(DK))))


def _peratom_kernel(c24_ref, wcat_ref, out_ref):
    out_ref[...] = jnp.dot(c24_ref[...], wcat_ref[...],
                           preferred_element_type=jnp.float32)


def _iota2(shape, dim):
    return jax.lax.broadcasted_iota(jnp.int32, shape, dim).astype(jnp.float32)


def _edge_kernel(coord_ref, veloc_ref, qw_ref,
                 ak_ref, akb_ref, av0_ref, avb0_ref, ave_ref, avbe_ref,
                 tk_ref, tv_ref,
                 mk1_ref, mkb1_ref, mv1_ref, mvb1_ref,
                 bk34_ref, bk34b_ref, bv34_ref, bv34b_ref,
                 u34_ref, u34b_ref, wout0_ref, w1o_ref,
                 out0_ref, out1_ref, attn_ref):
    i = pl.program_id(1)

    cb = coord_ref[0]
    vb = veloc_ref[0]
    ci = coord_ref[0, pl.ds(i, 1), :]
    vi = veloc_ref[0, pl.ds(i, 1), :]

    ce = cb - ci
    ve = vb - vi
    sumsq = jnp.sum(ce * ce, axis=1, keepdims=True)
    d = jnp.sqrt(sumsq + np.float32(1e-12))
    dot_ve = jnp.sum(ve * ce, axis=1, keepdims=True)
    inv2s3d = np.float32(-0.5 / np.sqrt(3.0)) / d
    s1a = sumsq * inv2s3d
    s1b = dot_ve * inv2s3d

    kvals = _iota2((1, DE), 1) * RSTEP
    diff = (d - kvals) * np.float32(1.0 / RSTEP)
    rbf = jnp.exp(-diff * diff) * np.float32(1.0 / 1.12)

    hk = jax.nn.silu(
        jnp.dot(rbf, mk1_ref[...], preferred_element_type=jnp.float32)
        + tk_ref[0, 0] + mkb1_ref[...])
    hv = jax.nn.silu(
        jnp.dot(rbf, mv1_ref[...], preferred_element_type=jnp.float32)
        + tv_ref[0, 0] + mvb1_ref[...])

    g34 = jnp.dot(hk, bk34_ref[...],
                  preferred_element_type=jnp.float32) + bk34b_ref[...]
    gi = jnp.dot(hk, ak_ref[0, 0],
                 preferred_element_type=jnp.float32) + akb_ref[0, 0]
    keyf = (gi + s1a * g34[:, :32] + s1b * g34[:, 32:]) * (1.0 / SQRT26)

    prod = qw_ref[0] * keyf
    sel = (_iota2((32, H), 0) * np.float32(1.0 / DK)).astype(jnp.int32)
    selm = (sel == jax.lax.broadcasted_iota(jnp.int32, (32, H), 1))
    a = jnp.dot(prod, selm.astype(jnp.float32),
                preferred_element_type=jnp.float32) * ASCL
    amax = jnp.max(a, axis=0, keepdims=True)
    ex = jnp.exp(a - amax)
    attn = ex / jnp.sum(ex, axis=0, keepdims=True)

    gv34 = jnp.dot(hv, bv34_ref[...],
                   preferred_element_type=jnp.float32) + bv34b_ref[...]
    gvi = jnp.dot(hv, av0_ref[0, 0],
                  preferred_element_type=jnp.float32) + avb0_ref[0, 0]
    val0 = (gvi + s1a * gv34[:, :32] + s1b * gv34[:, 32:]) * (1.0 / SQRT26)

    e1 = jnp.dot(hv, ave_ref[0, 0],
                 preferred_element_type=jnp.float32) + avbe_ref[0, 0]
    gu = jnp.dot(hv, u34_ref[...],
                 preferred_element_type=jnp.float32) + u34b_ref[...]
    coefc = e1 / d - np.float32(0.5) * gu[:, :DV0]
    u4h = np.float32(0.5) * gu[:, DV0:]

    r8 = ((_iota2((DV0, 24), 1) * np.float32(1.0 / 3.0)).astype(jnp.int32)
          == jax.lax.broadcasted_iota(jnp.int32, (DV0, 24), 0))
    r8 = r8.astype(jnp.float32)
    x_idx = jax.lax.broadcasted_iota(jnp.int32, (3, 24), 1)
    t3 = (x_idx - (x_idx * np.float32(1.0 / 3.0)).astype(jnp.int32) * 3
          == jax.lax.broadcasted_iota(jnp.int32, (3, 24), 0))
    t3 = t3.astype(jnp.float32)
    ce_t = jnp.dot(ce, t3, preferred_element_type=jnp.float32)
    ve_t = jnp.dot(ve, t3, preferred_element_type=jnp.float32)
    cc_r = jnp.dot(coefc, r8, preferred_element_type=jnp.float32)
    u4_r = jnp.dot(u4h, r8, preferred_element_type=jnp.float32)
    val1 = (cc_r * ce_t - u4_r * ve_t) * (1.0 / SQRT78)

    rep8 = ((_iota2((H, 32), 1) * np.float32(1.0 / DK)).astype(jnp.int32)
            == jax.lax.broadcasted_iota(jnp.int32, (H, 32), 0))
    rep6 = ((_iota2((H, 24), 1) * np.float32(1.0 / 6.0)).astype(jnp.int32)
            == jax.lax.broadcasted_iota(jnp.int32, (H, 24), 0))
    at32 = jnp.dot(attn, rep8.astype(jnp.float32),
                   preferred_element_type=jnp.float32)
    at24 = jnp.dot(attn, rep6.astype(jnp.float32),
                   preferred_element_type=jnp.float32)
    x0 = jnp.sum(at32 * val0, axis=0, keepdims=True)
    x1 = jnp.sum(at24 * val1, axis=0, keepdims=True)

    out0_ref[0, 0] = jnp.dot(x0, wout0_ref[...],
                             preferred_element_type=jnp.float32) * np.float32(
                                 1.0 / np.sqrt(32.0))
    out1_ref[0, 0] = jnp.dot(x1, w1o_ref[...],
                             preferred_element_type=jnp.float32)

    eye = (jax.lax.broadcasted_iota(jnp.int32, (N, N), 0)
           == jax.lax.broadcasted_iota(jnp.int32, (N, N), 1)).astype(jnp.float32)
    d_row = jnp.sum(d * eye, axis=0, keepdims=True)
    j_io = jax.lax.broadcasted_iota(jnp.int32, (N, N), 0)
    k_io = jax.lax.broadcasted_iota(jnp.int32, (N, N), 1)
    less = jnp.where(
        (d_row < d) | ((d_row == d) & (k_io < j_io)),
        np.float32(1.0), np.float32(0.0))
    rank = jnp.sum(less, axis=1, keepdims=True)
    rank_row = jnp.sum(rank * eye, axis=0, keepdims=True)
    perm = jnp.where(rank_row == _iota2((N, N), 0),
                     np.float32(1.0), np.float32(0.0))
    attn_ref[0, 0] = jnp.dot(perm, attn,
                             preferred_element_type=jnp.float32)


@jax.jit
def kernel(sp, coord, veloc, t, Wq, mkW1, mkb1, mkW2, mkb2,
           mvW1, mvb1, mvW2, mvb2, w_dot, Wout0e, Wout1o):
    f32 = jnp.float32

    wk_atom = mkW2[:, :768].reshape(64, 24, 32).transpose(1, 0, 2).reshape(24, 2048)
    bk_atom = mkb2[:768].reshape(24, 32)
    bk34 = mkW2[:, 768:832]
    bk34b = mkb2[768:832].reshape(1, 64)
    wv_atom0 = mvW2[:, :768].reshape(64, 24, 32).transpose(1, 0, 2).reshape(24, 2048)
    bv_atom0 = mvb2[:768].reshape(24, 32)
    bv34 = mvW2[:, 768:832]
    bv34b = mvb2[768:832].reshape(1, 64)
    wv_atome = mvW2[:, 832:1024].reshape(64, 24, 8).transpose(1, 0, 2).reshape(24, 512)
    bv_atome = mvb2[832:1024].reshape(24, 8)
    u34 = mvW2[:, 1024:1040]
    u34b = mvb2[1024:1040].reshape(1, 16)
    wq_fold = (Wq.reshape(24, H, DK) @ w_dot).reshape(24, 32) * f32(
        1.0 / np.sqrt(24.0))
    w1o_exp = jnp.einsum('ko,xy->kxoy', Wout1o,
                         jnp.eye(3, dtype=f32)).reshape(24, 6) * f32(
                             1.0 / np.sqrt(8.0))
    zpad = jnp.zeros((SP, 64), f32)
    tk_w = jnp.concatenate([zpad, mkW1[SP:, :]], 0)
    tv_w = jnp.concatenate([zpad, mvW1[SP:, :]], 0)
    wcat = jnp.concatenate([
        wk_atom, bk_atom, wv_atom0, bv_atom0, wv_atome, bv_atome,
        wq_fold, tk_w, tv_w], axis=1)

    c24 = jnp.concatenate([sp, t], -1).reshape(B * N, 24)

    pa = pl.pallas_call(
        _peratom_kernel,
        out_shape=jax.ShapeDtypeStruct((B * N, 4840), f32),
    )(c24, wcat)

    pa = pa.reshape(B, N, 4840)
    a_k = pa[..., 0:2048].reshape(B, N, 64, 32)
    akb = pa[..., 2048:2080].reshape(B, N, 1, 32)
    a_v0 = pa[..., 2080:4128].reshape(B, N, 64, 32)
    avb0 = pa[..., 4128:4160].reshape(B, N, 1, 32)
    a_ve = pa[..., 4160:4672].reshape(B, N, 64, 8)
    avbe = pa[..., 4672:4680].reshape(B, N, 1, 8)
    qw = pa[..., 4680:4712]
    tk = pa[..., 4712:4776].reshape(B, N, 1, 64)
    tv = pa[..., 4776:4840].reshape(B, N, 1, 64)

    full2 = lambda arr: pl.BlockSpec(arr.shape, lambda b, i: (0, 0))
    perb = lambda shp: pl.BlockSpec((1,) + shp[1:], lambda b, i: (b, 0, 0))
    peri = lambda shp: pl.BlockSpec((1, 1) + shp[2:],
                                    lambda b, i: (b, i, 0, 0))

    mkb1r = mkb1.reshape(1, 64)
    mvb1r = mvb1.reshape(1, 64)

    out0, out1, attn_s = pl.pallas_call(
        _edge_kernel,
        grid=(B, N),
        in_specs=[
            perb(coord.shape), perb(veloc.shape), perb(qw.shape),
            peri(a_k.shape), peri(akb.shape), peri(a_v0.shape),
            peri(avb0.shape), peri(a_ve.shape), peri(avbe.shape),
            peri(tk.shape), peri(tv.shape),
            full2(mkW1[:SP]), full2(mkb1r), full2(mvW1[:SP]), full2(mvb1r),
            full2(bk34), full2(bk34b), full2(bv34), full2(bv34b),
            full2(u34), full2(u34b), full2(Wout0e), full2(w1o_exp),
        ],
        out_specs=[
            pl.BlockSpec((1, 1, 1, 24), lambda b, i: (b, i, 0, 0)),
            pl.BlockSpec((1, 1, 1, 6), lambda b, i: (b, i, 0, 0)),
            pl.BlockSpec((1, 1, N, H), lambda b, i: (b, i, 0, 0)),
        ],
        out_shape=[
            jax.ShapeDtypeStruct((B, N, 1, 24), f32),
            jax.ShapeDtypeStruct((B, N, 1, 6), f32),
            jax.ShapeDtypeStruct((B, N, N, H), f32),
        ],
    )(coord, veloc, qw, a_k, akb, a_v0, avb0, a_ve, avbe, tk, tv,
      mkW1[:SP], mkb1r, mvW1[:SP], mvb1r, bk34, bk34b, bv34, bv34b,
      u34, u34b, Wout0e, w1o_exp)

    out0 = out0.reshape(B, N, 24)
    out1 = out1.reshape(B, N, 6)
    sp_o = out0[..., :SP]
    t_o = out0[..., SP:]
    coord_o = out1[..., :3]
    veloc_o = out1[..., 3:]
    attn = attn_s.transpose(0, 3, 1, 2)[..., None]
    return (sp_o, coord_o, veloc_o, t_o, attn)
